# baseline (speedup 1.0000x reference)
import jax
import jax.numpy as jnp
from jax import lax
from jax.experimental import pallas as pl
from jax.experimental.pallas import tpu as pltpu
from jax.experimental.pallas import tpu_sc as plsc

_ROWS = 1024
_NOUT = 50
_STRIDE = 2000
_NPAD = 64
_NCHUNK = 4
_CW = _ROWS // _NCHUNK


def _gather_body(xt_hbm, out_hbm, rows_v, sem):
    wid = lax.axis_index("s")
    g = wid // _NCHUNK
    ch = wid % _NCHUNK
    idx = jnp.minimum(lax.iota(jnp.int32, 16) + g * 16, _NOUT - 1) * _STRIDE
    pltpu.async_copy(
        xt_hbm.at[idx, pl.ds(ch * _CW, _CW)], rows_v, sem).wait()
    pltpu.sync_copy(
        rows_v, out_hbm.at[pl.ds(g * 16, 16), pl.ds(ch * _CW, _CW)])


@jax.jit
def kernel(inputs):
    xt = inputs.T
    k = pl.kernel(
        _gather_body,
        out_type=jax.ShapeDtypeStruct((_NPAD, _ROWS), jnp.float32),
        mesh=plsc.VectorSubcoreMesh(
            core_axis_name="c", subcore_axis_name="s", num_cores=1),
        scratch_types=[
            pltpu.VMEM((16, _CW), jnp.float32),
            pltpu.SemaphoreType.DMA,
        ],
    )
    return k(xt)[:_NOUT].T
